# baseline (speedup 1.0000x reference)
import functools

import jax
import jax.numpy as jnp
from jax import lax
from jax.experimental import pallas as pl
from jax.experimental.pallas import tpu as pltpu
from jax.experimental.pallas import tpu_sc as plsc

N_NODES = 50000
N_EDGES = 800000
F_IN = 100
HID = 32
N_CLS = 2

PAD_IDX = N_NODES
NTAB = 50008
CHUNK = 256
BLK = 1064
NBLK = 47
ROWS_PER_TILE = 3128
LAST_ROWS = NTAB - 15 * ROWS_PER_TILE
NSLOT = 3
LAG_G = 1
LAG_S = 2


def _sc_info():
    try:
        info = plsc.get_sparse_core_info()
        return info.num_cores, info.num_subcores
    except Exception:
        return 2, 16


def _sc_degree(eidx, zeros1, ones):
    nc, ns = _sc_info()
    nch = eidx.shape[1]
    mesh = plsc.VectorSubcoreMesh(core_axis_name="c", subcore_axis_name="s")

    @functools.partial(
        pl.kernel,
        out_type=jax.ShapeDtypeStruct((nc, NTAB, 8), jnp.float32),
        mesh=mesh,
        compiler_params=pltpu.CompilerParams(use_tc_tiling_on_sc=False),
        scratch_types=[
            pltpu.VMEM((nch, 2, CHUNK), jnp.int32),
            pltpu.VMEM((CHUNK, 8), jnp.float32),
            pltpu.VMEM_SHARED((NTAB, 8), jnp.float32),
            pltpu.SemaphoreType.DMA,
        ],
    )
    def k(col_h, z_h, ones_h, out_h, cidx_v, ones_v, acc, dsem):
        cid = lax.axis_index("c")
        sid = lax.axis_index("s")
        wid = sid * nc + cid
        @pl.when(sid < ns - 1)
        def _():
            pltpu.sync_copy(z_h.at[pl.ds(sid * ROWS_PER_TILE, ROWS_PER_TILE)],
                            acc.at[pl.ds(sid * ROWS_PER_TILE, ROWS_PER_TILE)])
        @pl.when(sid == ns - 1)
        def _():
            pltpu.sync_copy(z_h.at[pl.ds((ns - 1) * ROWS_PER_TILE, LAST_ROWS)],
                            acc.at[pl.ds((ns - 1) * ROWS_PER_TILE, LAST_ROWS)])
        pltpu.sync_copy(col_h.at[wid], cidx_v)
        pltpu.sync_copy(ones_h, ones_v)
        plsc.subcore_barrier()

        @pl.loop(0, nch)
        def _(c):
            pltpu.async_copy(ones_v, acc.at[cidx_v.at[c, 1]], dsem, add=True)

        @pl.loop(0, nch)
        def _(c):
            pltpu.make_async_copy(ones_v, acc.at[cidx_v.at[c, 1]], dsem).wait()

        plsc.subcore_barrier()
        @pl.when(sid < ns - 1)
        def _():
            pltpu.sync_copy(acc.at[pl.ds(sid * ROWS_PER_TILE, ROWS_PER_TILE)],
                            out_h.at[cid, pl.ds(sid * ROWS_PER_TILE, ROWS_PER_TILE)])
        @pl.when(sid == ns - 1)
        def _():
            pltpu.sync_copy(acc.at[pl.ds((ns - 1) * ROWS_PER_TILE, LAST_ROWS)],
                            out_h.at[cid, pl.ds((ns - 1) * ROWS_PER_TILE, LAST_ROWS)])

    return k(eidx, zeros1, ones)


def _sc_seg_sum(table, eidx, zeros2):
    nc, ns = _sc_info()
    nch = eidx.shape[1]
    mesh = plsc.VectorSubcoreMesh(core_axis_name="c", subcore_axis_name="s")

    @functools.partial(
        pl.kernel,
        out_type=jax.ShapeDtypeStruct((nc, NTAB, HID), jnp.float32),
        mesh=mesh,
        compiler_params=pltpu.CompilerParams(use_tc_tiling_on_sc=False),
        scratch_types=[
            pltpu.VMEM((NSLOT, 2, CHUNK), jnp.int32),
            pltpu.VMEM((NSLOT, CHUNK, HID), jnp.float32),
            pltpu.VMEM_SHARED((NTAB, HID), jnp.float32),
            pltpu.SemaphoreType.DMA((NSLOT,)),
            pltpu.SemaphoreType.DMA((NSLOT,)),
            pltpu.SemaphoreType.DMA((NSLOT,)),
        ],
    )
    def k(tab_h, ei_h, z_h, out_h, idx_v, data_v, acc, isem, gsem, ssem):
        cid = lax.axis_index("c")
        sid = lax.axis_index("s")
        wid = sid * nc + cid
        @pl.when(sid < ns - 1)
        def _():
            pltpu.sync_copy(z_h.at[pl.ds(sid * ROWS_PER_TILE, ROWS_PER_TILE)],
                            acc.at[pl.ds(sid * ROWS_PER_TILE, ROWS_PER_TILE)])
        @pl.when(sid == ns - 1)
        def _():
            pltpu.sync_copy(z_h.at[pl.ds((ns - 1) * ROWS_PER_TILE, LAST_ROWS)],
                            acc.at[pl.ds((ns - 1) * ROWS_PER_TILE, LAST_ROWS)])
        plsc.subcore_barrier()

        @pl.loop(0, nch + LAG_S)
        def _(c):
            @pl.when(c < nch)
            def _():
                j = lax.rem(c, NSLOT)
                @pl.when(c >= NSLOT)
                def _():
                    pltpu.make_async_copy(data_v.at[j], acc.at[idx_v.at[j, 1]],
                                          ssem.at[j]).wait()
                pltpu.async_copy(ei_h.at[wid, c], idx_v.at[j], isem.at[j])
            @pl.when(jnp.logical_and(c >= LAG_G, c < nch + LAG_G))
            def _():
                g = c - LAG_G
                j = lax.rem(g, NSLOT)
                pltpu.make_async_copy(ei_h.at[wid, 0], idx_v.at[j],
                                      isem.at[j]).wait()
                pltpu.async_copy(tab_h.at[idx_v.at[j, 0]], data_v.at[j],
                                 gsem.at[j])
            @pl.when(c >= LAG_S)
            def _():
                s = c - LAG_S
                j = lax.rem(s, NSLOT)
                pltpu.make_async_copy(tab_h.at[idx_v.at[j, 0]], data_v.at[j],
                                      gsem.at[j]).wait()
                pltpu.async_copy(data_v.at[j], acc.at[idx_v.at[j, 1]],
                                 ssem.at[j], add=True)

        @pl.loop(0, NSLOT)
        def _(j):
            pltpu.make_async_copy(data_v.at[j], acc.at[idx_v.at[j, 1]],
                                  ssem.at[j]).wait()

        plsc.subcore_barrier()
        @pl.when(sid < ns - 1)
        def _():
            pltpu.sync_copy(acc.at[pl.ds(sid * ROWS_PER_TILE, ROWS_PER_TILE)],
                            out_h.at[cid, pl.ds(sid * ROWS_PER_TILE, ROWS_PER_TILE)])
        @pl.when(sid == ns - 1)
        def _():
            pltpu.sync_copy(acc.at[pl.ds((ns - 1) * ROWS_PER_TILE, LAST_ROWS)],
                            out_h.at[cid, pl.ds((ns - 1) * ROWS_PER_TILE, LAST_ROWS)])

    return k(table, eidx, zeros2)


def _tc_mm(x, wcat):
    def body(x_ref, w_ref, y_ref):
        y_ref[...] = jnp.dot(x_ref[...], w_ref[...],
                             preferred_element_type=jnp.float32)
    return pl.pallas_call(
        body,
        grid=(10,),
        in_specs=[pl.BlockSpec((5000, F_IN), lambda i: (i, 0)),
                  pl.BlockSpec((F_IN, 2 * HID), lambda i: (0, 0))],
        out_specs=pl.BlockSpec((5000, 2 * HID), lambda i: (i, 0)),
        out_shape=jax.ShapeDtypeStruct((N_NODES, 2 * HID), jnp.float32),
    )(x, wcat)


def _tc_scale(degp, yp):
    def body(d_ref, y_ref, dinv_ref, t1_ref):
        deg = d_ref[0, :, 0:1] + d_ref[1, :, 0:1]
        dinv = jnp.where(deg > 0, lax.rsqrt(jnp.maximum(deg, 1.0)), 0.0)
        dinv_ref[...] = dinv
        t1_ref[...] = y_ref[:, :HID] * dinv
    return pl.pallas_call(
        body,
        grid=(NBLK,),
        in_specs=[pl.BlockSpec((2, BLK, 8), lambda i: (0, i, 0)),
                  pl.BlockSpec((BLK, 2 * HID), lambda i: (i, 0))],
        out_specs=[pl.BlockSpec((BLK, 1), lambda i: (i, 0)),
                   pl.BlockSpec((BLK, HID), lambda i: (i, 0))],
        out_shape=[jax.ShapeDtypeStruct((NTAB, 1), jnp.float32),
                   jax.ShapeDtypeStruct((NTAB, HID), jnp.float32)],
    )(degp, yp)


def _tc_layer1(p1, dinv, yp, b1):
    def body(p_ref, d_ref, y_ref, b_ref, h_ref, t2_ref):
        s = (p_ref[0] + p_ref[1]) * d_ref[...]
        h = jnp.maximum(s + y_ref[:, HID:] + b_ref[...], 0.0)
        h_ref[...] = h
        t2_ref[...] = h * d_ref[...]
    return pl.pallas_call(
        body,
        grid=(NBLK,),
        in_specs=[pl.BlockSpec((2, BLK, HID), lambda i: (0, i, 0)),
                  pl.BlockSpec((BLK, 1), lambda i: (i, 0)),
                  pl.BlockSpec((BLK, 2 * HID), lambda i: (i, 0)),
                  pl.BlockSpec((1, HID), lambda i: (0, 0))],
        out_specs=[pl.BlockSpec((BLK, HID), lambda i: (i, 0)),
                   pl.BlockSpec((BLK, HID), lambda i: (i, 0))],
        out_shape=[jax.ShapeDtypeStruct((NTAB, HID), jnp.float32),
                   jax.ShapeDtypeStruct((NTAB, HID), jnp.float32)],
    )(p1, dinv, yp, b1)


def _tc_layer2(p2, dinv, hp, w2, v2, b2):
    def body(p_ref, d_ref, h_ref, w_ref, v_ref, b_ref, o_ref):
        z = (p_ref[0] + p_ref[1]) * d_ref[...]
        o = jnp.dot(z, w_ref[...], preferred_element_type=jnp.float32)
        o += jnp.dot(h_ref[...], v_ref[...], preferred_element_type=jnp.float32)
        o = jnp.maximum(o + b_ref[...], 0.0)
        m = jnp.max(o, axis=1, keepdims=True)
        lse = m + jnp.log(jnp.sum(jnp.exp(o - m), axis=1, keepdims=True))
        o_ref[...] = o - lse
    return pl.pallas_call(
        body,
        grid=(50,),
        in_specs=[pl.BlockSpec((2, 1000, HID), lambda i: (0, i, 0)),
                  pl.BlockSpec((1000, 1), lambda i: (i, 0)),
                  pl.BlockSpec((1000, HID), lambda i: (i, 0)),
                  pl.BlockSpec((HID, N_CLS), lambda i: (0, 0)),
                  pl.BlockSpec((HID, N_CLS), lambda i: (0, 0)),
                  pl.BlockSpec((1, N_CLS), lambda i: (0, 0))],
        out_specs=pl.BlockSpec((1000, N_CLS), lambda i: (i, 0)),
        out_shape=jax.ShapeDtypeStruct((N_NODES, N_CLS), jnp.float32),
    )(p2, dinv, hp, w2, v2, b2)


def kernel(x, edge_index, W1, V1, b1, W2, V2, b2):
    nc, ns = _sc_info()
    nw = nc * ns
    nch = -(-N_EDGES // (nw * CHUNK))
    e_pad = nw * nch * CHUNK
    npad = e_pad - N_EDGES

    eip = jnp.pad(edge_index, ((0, 0), (0, npad)), constant_values=PAD_IDX)
    eidx = jnp.transpose(eip.reshape(2, nw, nch, CHUNK), (1, 2, 0, 3))

    zeros1 = jnp.zeros((NTAB, 8), jnp.float32)
    zeros2 = jnp.zeros((NTAB, HID), jnp.float32)
    ones = jnp.ones((CHUNK, 8), jnp.float32)
    wcat = jnp.concatenate([W1, V1], axis=1)

    degp = _sc_degree(eidx, zeros1, ones)
    y = _tc_mm(x, wcat)
    yp = jnp.concatenate([y, jnp.zeros((NTAB - N_NODES, 2 * HID), jnp.float32)])
    dinv, t1p = _tc_scale(degp, yp)
    p1 = _sc_seg_sum(t1p, eidx, zeros2)
    hp, t2p = _tc_layer1(p1, dinv, yp, b1.reshape(1, HID))
    p2 = _sc_seg_sum(t2p, eidx, zeros2)
    outp = _tc_layer2(p2, dinv, hp, W2, V2, b2.reshape(1, N_CLS))
    return outp[:N_NODES]


# speedup vs baseline: 1.0494x; 1.0494x over previous
import functools

import jax
import jax.numpy as jnp
from jax import lax
from jax.experimental import pallas as pl
from jax.experimental.pallas import tpu as pltpu
from jax.experimental.pallas import tpu_sc as plsc

N_NODES = 50000
N_EDGES = 800000
F_IN = 100
HID = 32
N_CLS = 2

PAD_IDX = N_NODES
NTAB = 50008
CHUNK = 128
BLK = 1064
NBLK = 47
ROWS_PER_TILE = 3128
LAST_ROWS = NTAB - 15 * ROWS_PER_TILE
NSLOT = 7
LAG_G = 2
LAG_S = 4


def _sc_info():
    try:
        info = plsc.get_sparse_core_info()
        return info.num_cores, info.num_subcores
    except Exception:
        return 2, 16


def _sc_degree(eidx, zeros1, ones):
    nc, ns = _sc_info()
    nch = eidx.shape[1]
    mesh = plsc.VectorSubcoreMesh(core_axis_name="c", subcore_axis_name="s")

    @functools.partial(
        pl.kernel,
        out_type=jax.ShapeDtypeStruct((nc, NTAB, 8), jnp.float32),
        mesh=mesh,
        compiler_params=pltpu.CompilerParams(use_tc_tiling_on_sc=False),
        scratch_types=[
            pltpu.VMEM((nch, 2, CHUNK), jnp.int32),
            pltpu.VMEM((CHUNK, 8), jnp.float32),
            pltpu.VMEM_SHARED((NTAB, 8), jnp.float32),
            pltpu.SemaphoreType.DMA,
        ],
    )
    def k(col_h, z_h, ones_h, out_h, cidx_v, ones_v, acc, dsem):
        cid = lax.axis_index("c")
        sid = lax.axis_index("s")
        wid = sid * nc + cid
        @pl.when(sid < ns - 1)
        def _():
            pltpu.sync_copy(z_h.at[pl.ds(sid * ROWS_PER_TILE, ROWS_PER_TILE)],
                            acc.at[pl.ds(sid * ROWS_PER_TILE, ROWS_PER_TILE)])
        @pl.when(sid == ns - 1)
        def _():
            pltpu.sync_copy(z_h.at[pl.ds((ns - 1) * ROWS_PER_TILE, LAST_ROWS)],
                            acc.at[pl.ds((ns - 1) * ROWS_PER_TILE, LAST_ROWS)])
        pltpu.sync_copy(col_h.at[wid], cidx_v)
        pltpu.sync_copy(ones_h, ones_v)
        plsc.subcore_barrier()

        @pl.loop(0, nch)
        def _(c):
            pltpu.async_copy(ones_v, acc.at[cidx_v.at[c, 1]], dsem, add=True)

        @pl.loop(0, nch)
        def _(c):
            pltpu.make_async_copy(ones_v, acc.at[cidx_v.at[c, 1]], dsem).wait()

        plsc.subcore_barrier()
        @pl.when(sid < ns - 1)
        def _():
            pltpu.sync_copy(acc.at[pl.ds(sid * ROWS_PER_TILE, ROWS_PER_TILE)],
                            out_h.at[cid, pl.ds(sid * ROWS_PER_TILE, ROWS_PER_TILE)])
        @pl.when(sid == ns - 1)
        def _():
            pltpu.sync_copy(acc.at[pl.ds((ns - 1) * ROWS_PER_TILE, LAST_ROWS)],
                            out_h.at[cid, pl.ds((ns - 1) * ROWS_PER_TILE, LAST_ROWS)])

    return k(eidx, zeros1, ones)


def _sc_seg_sum(table, eidx, zeros2):
    nc, ns = _sc_info()
    nch = eidx.shape[1]
    mesh = plsc.VectorSubcoreMesh(core_axis_name="c", subcore_axis_name="s")

    @functools.partial(
        pl.kernel,
        out_type=jax.ShapeDtypeStruct((nc, NTAB, HID), jnp.float32),
        mesh=mesh,
        compiler_params=pltpu.CompilerParams(use_tc_tiling_on_sc=False),
        scratch_types=[
            pltpu.VMEM((NSLOT, 2, CHUNK), jnp.int32),
            pltpu.VMEM((NSLOT, CHUNK, HID), jnp.float32),
            pltpu.VMEM_SHARED((NTAB, HID), jnp.float32),
            pltpu.SemaphoreType.DMA((NSLOT,)),
            pltpu.SemaphoreType.DMA((NSLOT,)),
            pltpu.SemaphoreType.DMA((NSLOT,)),
        ],
    )
    def k(tab_h, ei_h, z_h, out_h, idx_v, data_v, acc, isem, gsem, ssem):
        cid = lax.axis_index("c")
        sid = lax.axis_index("s")
        wid = sid * nc + cid
        @pl.when(sid < ns - 1)
        def _():
            pltpu.sync_copy(z_h.at[pl.ds(sid * ROWS_PER_TILE, ROWS_PER_TILE)],
                            acc.at[pl.ds(sid * ROWS_PER_TILE, ROWS_PER_TILE)])
        @pl.when(sid == ns - 1)
        def _():
            pltpu.sync_copy(z_h.at[pl.ds((ns - 1) * ROWS_PER_TILE, LAST_ROWS)],
                            acc.at[pl.ds((ns - 1) * ROWS_PER_TILE, LAST_ROWS)])
        plsc.subcore_barrier()

        @pl.loop(0, nch + LAG_S)
        def _(c):
            @pl.when(c < nch)
            def _():
                j = lax.rem(c, NSLOT)
                @pl.when(c >= NSLOT)
                def _():
                    pltpu.make_async_copy(data_v.at[j], acc.at[idx_v.at[j, 1]],
                                          ssem.at[j]).wait()
                pltpu.async_copy(ei_h.at[wid, c], idx_v.at[j], isem.at[j])
            @pl.when(jnp.logical_and(c >= LAG_G, c < nch + LAG_G))
            def _():
                g = c - LAG_G
                j = lax.rem(g, NSLOT)
                pltpu.make_async_copy(ei_h.at[wid, 0], idx_v.at[j],
                                      isem.at[j]).wait()
                pltpu.async_copy(tab_h.at[idx_v.at[j, 0]], data_v.at[j],
                                 gsem.at[j])
            @pl.when(c >= LAG_S)
            def _():
                s = c - LAG_S
                j = lax.rem(s, NSLOT)
                pltpu.make_async_copy(tab_h.at[idx_v.at[j, 0]], data_v.at[j],
                                      gsem.at[j]).wait()
                pltpu.async_copy(data_v.at[j], acc.at[idx_v.at[j, 1]],
                                 ssem.at[j], add=True)

        @pl.loop(0, NSLOT)
        def _(j):
            pltpu.make_async_copy(data_v.at[j], acc.at[idx_v.at[j, 1]],
                                  ssem.at[j]).wait()

        plsc.subcore_barrier()
        @pl.when(sid < ns - 1)
        def _():
            pltpu.sync_copy(acc.at[pl.ds(sid * ROWS_PER_TILE, ROWS_PER_TILE)],
                            out_h.at[cid, pl.ds(sid * ROWS_PER_TILE, ROWS_PER_TILE)])
        @pl.when(sid == ns - 1)
        def _():
            pltpu.sync_copy(acc.at[pl.ds((ns - 1) * ROWS_PER_TILE, LAST_ROWS)],
                            out_h.at[cid, pl.ds((ns - 1) * ROWS_PER_TILE, LAST_ROWS)])

    return k(table, eidx, zeros2)


def _tc_mm(x, wcat):
    def body(x_ref, w_ref, y_ref):
        y_ref[...] = jnp.dot(x_ref[...], w_ref[...],
                             preferred_element_type=jnp.float32)
    return pl.pallas_call(
        body,
        grid=(10,),
        in_specs=[pl.BlockSpec((5000, F_IN), lambda i: (i, 0)),
                  pl.BlockSpec((F_IN, 2 * HID), lambda i: (0, 0))],
        out_specs=pl.BlockSpec((5000, 2 * HID), lambda i: (i, 0)),
        out_shape=jax.ShapeDtypeStruct((N_NODES, 2 * HID), jnp.float32),
    )(x, wcat)


def _tc_scale(degp, yp):
    def body(d_ref, y_ref, dinv_ref, t1_ref):
        deg = d_ref[0, :, 0:1] + d_ref[1, :, 0:1]
        dinv = jnp.where(deg > 0, lax.rsqrt(jnp.maximum(deg, 1.0)), 0.0)
        dinv_ref[...] = dinv
        t1_ref[...] = y_ref[:, :HID] * dinv
    return pl.pallas_call(
        body,
        grid=(NBLK,),
        in_specs=[pl.BlockSpec((2, BLK, 8), lambda i: (0, i, 0)),
                  pl.BlockSpec((BLK, 2 * HID), lambda i: (i, 0))],
        out_specs=[pl.BlockSpec((BLK, 1), lambda i: (i, 0)),
                   pl.BlockSpec((BLK, HID), lambda i: (i, 0))],
        out_shape=[jax.ShapeDtypeStruct((NTAB, 1), jnp.float32),
                   jax.ShapeDtypeStruct((NTAB, HID), jnp.float32)],
    )(degp, yp)


def _tc_layer1(p1, dinv, yp, b1):
    def body(p_ref, d_ref, y_ref, b_ref, h_ref, t2_ref):
        s = (p_ref[0] + p_ref[1]) * d_ref[...]
        h = jnp.maximum(s + y_ref[:, HID:] + b_ref[...], 0.0)
        h_ref[...] = h
        t2_ref[...] = h * d_ref[...]
    return pl.pallas_call(
        body,
        grid=(NBLK,),
        in_specs=[pl.BlockSpec((2, BLK, HID), lambda i: (0, i, 0)),
                  pl.BlockSpec((BLK, 1), lambda i: (i, 0)),
                  pl.BlockSpec((BLK, 2 * HID), lambda i: (i, 0)),
                  pl.BlockSpec((1, HID), lambda i: (0, 0))],
        out_specs=[pl.BlockSpec((BLK, HID), lambda i: (i, 0)),
                   pl.BlockSpec((BLK, HID), lambda i: (i, 0))],
        out_shape=[jax.ShapeDtypeStruct((NTAB, HID), jnp.float32),
                   jax.ShapeDtypeStruct((NTAB, HID), jnp.float32)],
    )(p1, dinv, yp, b1)


def _tc_layer2(p2, dinv, hp, w2, v2, b2):
    def body(p_ref, d_ref, h_ref, w_ref, v_ref, b_ref, o_ref):
        z = (p_ref[0] + p_ref[1]) * d_ref[...]
        o = jnp.dot(z, w_ref[...], preferred_element_type=jnp.float32)
        o += jnp.dot(h_ref[...], v_ref[...], preferred_element_type=jnp.float32)
        o = jnp.maximum(o + b_ref[...], 0.0)
        m = jnp.max(o, axis=1, keepdims=True)
        lse = m + jnp.log(jnp.sum(jnp.exp(o - m), axis=1, keepdims=True))
        o_ref[...] = o - lse
    return pl.pallas_call(
        body,
        grid=(50,),
        in_specs=[pl.BlockSpec((2, 1000, HID), lambda i: (0, i, 0)),
                  pl.BlockSpec((1000, 1), lambda i: (i, 0)),
                  pl.BlockSpec((1000, HID), lambda i: (i, 0)),
                  pl.BlockSpec((HID, N_CLS), lambda i: (0, 0)),
                  pl.BlockSpec((HID, N_CLS), lambda i: (0, 0)),
                  pl.BlockSpec((1, N_CLS), lambda i: (0, 0))],
        out_specs=pl.BlockSpec((1000, N_CLS), lambda i: (i, 0)),
        out_shape=jax.ShapeDtypeStruct((N_NODES, N_CLS), jnp.float32),
    )(p2, dinv, hp, w2, v2, b2)


def kernel(x, edge_index, W1, V1, b1, W2, V2, b2):
    nc, ns = _sc_info()
    nw = nc * ns
    nch = -(-N_EDGES // (nw * CHUNK))
    e_pad = nw * nch * CHUNK
    npad = e_pad - N_EDGES

    eip = jnp.pad(edge_index, ((0, 0), (0, npad)), constant_values=PAD_IDX)
    eidx = jnp.transpose(eip.reshape(2, nw, nch, CHUNK), (1, 2, 0, 3))

    zeros1 = jnp.zeros((NTAB, 8), jnp.float32)
    zeros2 = jnp.zeros((NTAB, HID), jnp.float32)
    ones = jnp.ones((CHUNK, 8), jnp.float32)
    wcat = jnp.concatenate([W1, V1], axis=1)

    degp = _sc_degree(eidx, zeros1, ones)
    y = _tc_mm(x, wcat)
    yp = jnp.concatenate([y, jnp.zeros((NTAB - N_NODES, 2 * HID), jnp.float32)])
    dinv, t1p = _tc_scale(degp, yp)
    p1 = _sc_seg_sum(t1p, eidx, zeros2)
    hp, t2p = _tc_layer1(p1, dinv, yp, b1.reshape(1, HID))
    p2 = _sc_seg_sum(t2p, eidx, zeros2)
    outp = _tc_layer2(p2, dinv, hp, W2, V2, b2.reshape(1, N_CLS))
    return outp[:N_NODES]


# speedup vs baseline: 1.4155x; 1.3488x over previous
import functools

import jax
import jax.numpy as jnp
from jax import lax
from jax.experimental import pallas as pl
from jax.experimental.pallas import tpu as pltpu
from jax.experimental.pallas import tpu_sc as plsc

N_NODES = 50000
N_EDGES = 800000
F_IN = 100
HID = 32
N_CLS = 2

PAD_IDX = N_NODES
NTAB = 50016
PACKED = NTAB * HID // 128
ROWS_PER_TILE = 3128
LAST_ROWS = NTAB - 15 * ROWS_PER_TILE
CHUNK = 128
NSLOT = 7
LAG_G = 2
LAG_S = 4
NB = 3
BROW = PACKED // NB
NODE_B = NTAB // NB


def _sc_info():
    try:
        info = plsc.get_sparse_core_info()
        return info.num_cores, info.num_subcores
    except Exception:
        return 2, 16


def _zero_acc(z_h, acc, sid, ns):
    @pl.when(sid < ns - 1)
    def _():
        pltpu.sync_copy(z_h.at[pl.ds(sid * ROWS_PER_TILE, ROWS_PER_TILE)],
                        acc.at[pl.ds(sid * ROWS_PER_TILE, ROWS_PER_TILE)])
    @pl.when(sid == ns - 1)
    def _():
        pltpu.sync_copy(z_h.at[pl.ds((ns - 1) * ROWS_PER_TILE, LAST_ROWS)],
                        acc.at[pl.ds((ns - 1) * ROWS_PER_TILE, LAST_ROWS)])


def _write_acc(acc, out_h, cid, sid, ns):
    @pl.when(sid < ns - 1)
    def _():
        pltpu.sync_copy(acc.at[pl.ds(sid * ROWS_PER_TILE, ROWS_PER_TILE)],
                        out_h.at[cid, pl.ds(sid * ROWS_PER_TILE, ROWS_PER_TILE)])
    @pl.when(sid == ns - 1)
    def _():
        pltpu.sync_copy(acc.at[pl.ds((ns - 1) * ROWS_PER_TILE, LAST_ROWS)],
                        out_h.at[cid, pl.ds((ns - 1) * ROWS_PER_TILE, LAST_ROWS)])


def _sc_degree(eidx, zeros2, ones):
    nc, ns = _sc_info()
    nch = eidx.shape[1]
    mesh = plsc.VectorSubcoreMesh(core_axis_name="c", subcore_axis_name="s")

    @functools.partial(
        pl.kernel,
        out_type=jax.ShapeDtypeStruct((nc, NTAB, HID), jnp.float32),
        mesh=mesh,
        compiler_params=pltpu.CompilerParams(use_tc_tiling_on_sc=False),
        scratch_types=[
            pltpu.VMEM((NSLOT, 2, CHUNK), jnp.int32),
            pltpu.VMEM((CHUNK, HID), jnp.float32),
            pltpu.VMEM_SHARED((NTAB, HID), jnp.float32),
            pltpu.SemaphoreType.DMA((NSLOT,)),
            pltpu.SemaphoreType.DMA((NSLOT,)),
        ],
    )
    def k(ei_h, z_h, ones_h, out_h, idx_v, ones_v, acc, isem, ssem):
        cid = lax.axis_index("c")
        sid = lax.axis_index("s")
        wid = sid * nc + cid
        _zero_acc(z_h, acc, sid, ns)
        pltpu.sync_copy(ones_h, ones_v)
        plsc.subcore_barrier()

        @pl.loop(0, nch + LAG_G)
        def _(c):
            @pl.when(c < nch)
            def _():
                j = lax.rem(c, NSLOT)
                @pl.when(c >= NSLOT)
                def _():
                    pltpu.make_async_copy(ones_v, acc.at[idx_v.at[j, 1]],
                                          ssem.at[j]).wait()
                pltpu.async_copy(ei_h.at[wid, c], idx_v.at[j], isem.at[j])
            @pl.when(c >= LAG_G)
            def _():
                s = c - LAG_G
                j = lax.rem(s, NSLOT)
                pltpu.make_async_copy(ei_h.at[wid, 0], idx_v.at[j],
                                      isem.at[j]).wait()
                pltpu.async_copy(ones_v, acc.at[idx_v.at[j, 1]],
                                 ssem.at[j], add=True)

        @pl.loop(0, NSLOT)
        def _(j):
            pltpu.make_async_copy(ones_v, acc.at[idx_v.at[j, 1]],
                                  ssem.at[j]).wait()

        plsc.subcore_barrier()
        _write_acc(acc, out_h, cid, sid, ns)

    return k(eidx, zeros2, ones)


def _sc_seg_sum(table, eidx, zeros2):
    nc, ns = _sc_info()
    nch = eidx.shape[1]
    mesh = plsc.VectorSubcoreMesh(core_axis_name="c", subcore_axis_name="s")

    @functools.partial(
        pl.kernel,
        out_type=jax.ShapeDtypeStruct((nc, NTAB, HID), jnp.float32),
        mesh=mesh,
        compiler_params=pltpu.CompilerParams(use_tc_tiling_on_sc=False),
        scratch_types=[
            pltpu.VMEM((NSLOT, 2, CHUNK), jnp.int32),
            pltpu.VMEM((NSLOT, CHUNK, HID), jnp.float32),
            pltpu.VMEM_SHARED((NTAB, HID), jnp.float32),
            pltpu.SemaphoreType.DMA((NSLOT,)),
            pltpu.SemaphoreType.DMA((NSLOT,)),
            pltpu.SemaphoreType.DMA((NSLOT,)),
        ],
    )
    def k(tab_h, ei_h, z_h, out_h, idx_v, data_v, acc, isem, gsem, ssem):
        cid = lax.axis_index("c")
        sid = lax.axis_index("s")
        wid = sid * nc + cid
        _zero_acc(z_h, acc, sid, ns)
        plsc.subcore_barrier()

        @pl.loop(0, nch + LAG_S)
        def _(c):
            @pl.when(c < nch)
            def _():
                j = lax.rem(c, NSLOT)
                @pl.when(c >= NSLOT)
                def _():
                    pltpu.make_async_copy(data_v.at[j], acc.at[idx_v.at[j, 1]],
                                          ssem.at[j]).wait()
                pltpu.async_copy(ei_h.at[wid, c], idx_v.at[j], isem.at[j])
            @pl.when(jnp.logical_and(c >= LAG_G, c < nch + LAG_G))
            def _():
                g = c - LAG_G
                j = lax.rem(g, NSLOT)
                pltpu.make_async_copy(ei_h.at[wid, 0], idx_v.at[j],
                                      isem.at[j]).wait()
                pltpu.async_copy(tab_h.at[idx_v.at[j, 0]], data_v.at[j],
                                 gsem.at[j])
            @pl.when(c >= LAG_S)
            def _():
                s = c - LAG_S
                j = lax.rem(s, NSLOT)
                pltpu.make_async_copy(tab_h.at[idx_v.at[j, 0]], data_v.at[j],
                                      gsem.at[j]).wait()
                pltpu.async_copy(data_v.at[j], acc.at[idx_v.at[j, 1]],
                                 ssem.at[j], add=True)

        @pl.loop(0, NSLOT)
        def _(j):
            pltpu.make_async_copy(data_v.at[j], acc.at[idx_v.at[j, 1]],
                                  ssem.at[j]).wait()

        plsc.subcore_barrier()
        _write_acc(acc, out_h, cid, sid, ns)

    return k(table, eidx, zeros2)


def _tc_in(x4, wblk, degp):
    def body(x_ref, w_ref, d_ref, dinv_ref, t1_ref, r1_ref):
        y = jnp.dot(x_ref[...], w_ref[...], preferred_element_type=jnp.float32)
        deg = d_ref[0] + d_ref[1]
        dinv = jnp.where(deg > 0, lax.rsqrt(jnp.maximum(deg, 1.0)), 0.0)
        dinv_ref[...] = dinv
        t1_ref[...] = y[:, :128] * dinv
        r1_ref[...] = y[:, 128:]
    return pl.pallas_call(
        body,
        grid=(NB,),
        in_specs=[pl.BlockSpec((BROW, 4 * F_IN), lambda i: (i, 0)),
                  pl.BlockSpec((4 * F_IN, 256), lambda i: (0, 0)),
                  pl.BlockSpec((2, BROW, 128), lambda i: (0, i, 0))],
        out_specs=[pl.BlockSpec((BROW, 128), lambda i: (i, 0)),
                   pl.BlockSpec((BROW, 128), lambda i: (i, 0)),
                   pl.BlockSpec((BROW, 128), lambda i: (i, 0))],
        out_shape=[jax.ShapeDtypeStruct((PACKED, 128), jnp.float32),
                   jax.ShapeDtypeStruct((PACKED, 128), jnp.float32),
                   jax.ShapeDtypeStruct((PACKED, 128), jnp.float32)],
    )(x4, wblk, degp)


def _tc_layer1(p1, dinv, r1p, b1t):
    def body(p_ref, d_ref, r_ref, b_ref, h_ref, t2_ref):
        s = (p_ref[0] + p_ref[1]) * d_ref[...]
        h = jnp.maximum(s + r_ref[...] + b_ref[...], 0.0)
        h_ref[...] = h
        t2_ref[...] = h * d_ref[...]
    return pl.pallas_call(
        body,
        grid=(NB,),
        in_specs=[pl.BlockSpec((2, BROW, 128), lambda i: (0, i, 0)),
                  pl.BlockSpec((BROW, 128), lambda i: (i, 0)),
                  pl.BlockSpec((BROW, 128), lambda i: (i, 0)),
                  pl.BlockSpec((1, 128), lambda i: (0, 0))],
        out_specs=[pl.BlockSpec((BROW, 128), lambda i: (i, 0)),
                   pl.BlockSpec((BROW, 128), lambda i: (i, 0))],
        out_shape=[jax.ShapeDtypeStruct((PACKED, 128), jnp.float32),
                   jax.ShapeDtypeStruct((PACKED, 128), jnp.float32)],
    )(p1, dinv, r1p, b1t)


def _tc_layer2(p2, dinv, hp, w2blk, v2blk, b2t):
    def body(p_ref, d_ref, h_ref, w_ref, v_ref, b_ref, o_ref):
        z = (p_ref[0] + p_ref[1]) * d_ref[...]
        o = jnp.dot(z, w_ref[...], preferred_element_type=jnp.float32)
        o += jnp.dot(h_ref[...], v_ref[...], preferred_element_type=jnp.float32)
        o = jnp.maximum(o + b_ref[...], 0.0)
        left = jnp.concatenate([o[:, 1:], o[:, :1]], axis=1)
        right = jnp.concatenate([o[:, -1:], o[:, :-1]], axis=1)
        parity = lax.rem(lax.broadcasted_iota(jnp.int32, o.shape, 1), 2)
        partner = jnp.where(parity == 0, left, right)
        m = jnp.maximum(o, partner)
        lse = m + jnp.log(jnp.exp(o - m) + jnp.exp(partner - m))
        o_ref[...] = o - lse
    return pl.pallas_call(
        body,
        grid=(NB,),
        in_specs=[pl.BlockSpec((2, BROW, 128), lambda i: (0, i, 0)),
                  pl.BlockSpec((BROW, 128), lambda i: (i, 0)),
                  pl.BlockSpec((BROW, 128), lambda i: (i, 0)),
                  pl.BlockSpec((4 * HID, 4 * N_CLS), lambda i: (0, 0)),
                  pl.BlockSpec((4 * HID, 4 * N_CLS), lambda i: (0, 0)),
                  pl.BlockSpec((1, 4 * N_CLS), lambda i: (0, 0))],
        out_specs=pl.BlockSpec((BROW, 4 * N_CLS), lambda i: (i, 0)),
        out_shape=jax.ShapeDtypeStruct((PACKED, 4 * N_CLS), jnp.float32),
    )(p2, dinv, hp, w2blk, v2blk, b2t)


def kernel(x, edge_index, W1, V1, b1, W2, V2, b2):
    nc, ns = _sc_info()
    nw = nc * ns
    nch = -(-N_EDGES // (nw * CHUNK))
    e_pad = nw * nch * CHUNK
    npad = e_pad - N_EDGES

    eip = jnp.pad(edge_index, ((0, 0), (0, npad)), constant_values=PAD_IDX)
    eidx = jnp.transpose(eip.reshape(2, nw, nch, CHUNK), (1, 2, 0, 3))

    x4 = jnp.pad(x, ((0, NTAB - N_NODES), (0, 0))).reshape(PACKED, 4 * F_IN)
    zeros2 = jnp.zeros((NTAB, HID), jnp.float32)
    ones = jnp.ones((CHUNK, HID), jnp.float32)
    eye4 = jnp.eye(4, dtype=jnp.float32)
    w1blk = jnp.einsum("ij,ab->iajb", eye4, W1).reshape(4 * F_IN, 128)
    v1blk = jnp.einsum("ij,ab->iajb", eye4, V1).reshape(4 * F_IN, 128)
    wblk = jnp.concatenate([w1blk, v1blk], axis=1)
    w2blk = jnp.einsum("ij,ab->iajb", eye4, W2).reshape(4 * HID, 4 * N_CLS)
    v2blk = jnp.einsum("ij,ab->iajb", eye4, V2).reshape(4 * HID, 4 * N_CLS)
    b1t = jnp.tile(b1, 4).reshape(1, 128)
    b2t = jnp.tile(b2, 4).reshape(1, 4 * N_CLS)

    degp = _sc_degree(eidx, zeros2, ones)
    degpk = degp.reshape(nc, PACKED, 128)
    dinv, t1pk, r1pk = _tc_in(x4, wblk, degpk)
    p1 = _sc_seg_sum(t1pk.reshape(NTAB, HID), eidx, zeros2)
    hpk, t2pk = _tc_layer1(p1.reshape(nc, PACKED, 128), dinv, r1pk, b1t)
    p2 = _sc_seg_sum(t2pk.reshape(NTAB, HID), eidx, zeros2)
    outp = _tc_layer2(p2.reshape(nc, PACKED, 128), dinv, hpk, w2blk, v2blk,
                      b2t)
    return outp.reshape(NTAB, N_CLS)[:N_NODES]
